# baseline (speedup 1.0000x reference)
import functools

import jax
import jax.numpy as jnp
from jax import lax
from jax.experimental import pallas as pl
from jax.experimental.pallas import tpu as pltpu
from jax.experimental.pallas import tpu_sc as plsc

_L = 16
_NW = 32
_CH = 128

_LOG_COEFFS = (
    7.0376836292e-2, -1.1514610310e-1, 1.1676998740e-1,
    -1.2420140846e-1, 1.4249322787e-1, -1.6668057665e-1,
    1.9999714748e-1, -2.4999993993e-1, 3.3333331174e-1,
)
_LN2 = 0.6931471805599453
_SQRT2 = 1.41421356


def _softplus16(x):
    e = jnp.exp(jnp.minimum(x, 20.0))
    t = 1.0 + e
    i = lax.bitcast_convert_type(t, jnp.int32)
    ex = lax.shift_right_logical(i, 23) - 127
    m = lax.bitcast_convert_type((i & 0x7FFFFF) | 0x3F800000, jnp.float32)
    big = m > _SQRT2
    m = jnp.where(big, m * 0.5, m)
    exf = ex.astype(jnp.float32) + jnp.where(big, 1.0, 0.0)
    f = m - 1.0
    z = f * f
    p = jnp.full_like(f, _LOG_COEFFS[0])
    for c in _LOG_COEFFS[1:]:
        p = p * f + c
    logt = f * z * p - 0.5 * z + f + exf * _LN2
    return jnp.where(x > 20.0, x, logt)


def kernel(idxs, U, V):
    B = idxs.shape[0]
    D = U.shape[1]
    R = 2 * B
    rows_per_w = R // _NW
    nch = rows_per_w // _CH
    idx_i32 = idxs.astype(jnp.int32).reshape(_NW, rows_per_w)

    mesh = plsc.VectorSubcoreMesh(core_axis_name="c", subcore_axis_name="s")

    @functools.partial(
        pl.kernel,
        out_type=jax.ShapeDtypeStruct((R, 2 * D), jnp.float32),
        mesh=mesh,
        compiler_params=pltpu.CompilerParams(use_tc_tiling_on_sc=False),
        scratch_types=[
            pltpu.VMEM((rows_per_w,), jnp.int32),
            pltpu.VMEM((_CH, D), jnp.float32),
            pltpu.VMEM((_CH, D), jnp.float32),
            pltpu.VMEM((_CH, D), jnp.float32),
            pltpu.VMEM((_CH, D), jnp.float32),
            pltpu.VMEM((_CH, 2 * D), jnp.float32),
            pltpu.VMEM((_CH, 2 * D), jnp.float32),
            pltpu.SemaphoreType.DMA,
            pltpu.SemaphoreType.DMA,
            pltpu.SemaphoreType.DMA,
            pltpu.SemaphoreType.DMA,
        ],
    )
    def run(idx_hbm, u_hbm, v_hbm, out_hbm,
            idx_v, ubA, vbA, ubB, vbB, obA, obB, semA, semB, soA, soB):
        wid = lax.axis_index("s") * 2 + lax.axis_index("c")
        pltpu.sync_copy(idx_hbm.at[wid], idx_v)

        def fire(c, ub, vb, sem):
            ii = idx_v.at[pl.ds(c * _CH, _CH)]
            pltpu.async_copy(u_hbm.at[ii], ub, sem)
            pltpu.async_copy(v_hbm.at[ii], vb, sem)

        def consume(c, ub, vb, sem, ob, so):
            pltpu.make_async_copy(u_hbm.at[pl.ds(0, _CH)], ub, sem).wait()
            pltpu.make_async_copy(v_hbm.at[pl.ds(0, _CH)], vb, sem).wait()

            def row_body(r, _):
                for l in range(D // _L):
                    ob[r, pl.ds(l * _L, _L)] = ub[r, pl.ds(l * _L, _L)]
                    ob[r, pl.ds(D + l * _L, _L)] = _softplus16(
                        vb[r, pl.ds(l * _L, _L)])
                return 0

            lax.fori_loop(0, _CH, row_body, 0)
            out_base = wid * rows_per_w + c * _CH
            pltpu.async_copy(ob, out_hbm.at[pl.ds(out_base, _CH)], so).wait()

        fire(0, ubA, vbA, semA)

        def pair_body(g, _):
            c0 = 2 * g
            fire(c0 + 1, ubB, vbB, semB)
            consume(c0, ubA, vbA, semA, obA, soA)

            @pl.when(c0 + 2 < nch)
            def _():
                fire(c0 + 2, ubA, vbA, semA)

            consume(c0 + 1, ubB, vbB, semB, obB, soB)
            return 0

        lax.fori_loop(0, nch // 2, pair_body, 0)

    out = run(idx_i32, U, V)
    return out.reshape(B, 2, 2, D)


# speedup vs baseline: 2.0678x; 2.0678x over previous
import functools

import jax
import jax.numpy as jnp
from jax import lax
from jax.experimental import pallas as pl
from jax.experimental.pallas import tpu as pltpu
from jax.experimental.pallas import tpu_sc as plsc

_L = 16
_NW = 32
_BLK = 512
_RING = 128

_LOG_COEFFS = (
    7.0376836292e-2, -1.1514610310e-1, 1.1676998740e-1,
    -1.2420140846e-1, 1.4249322787e-1, -1.6668057665e-1,
    1.9999714748e-1, -2.4999993993e-1, 3.3333331174e-1,
)
_LN2 = 0.6931471805599453
_SQRT2 = 1.41421356


def _softplus16(x):
    e = jnp.exp(jnp.minimum(x, 20.0))
    t = 1.0 + e
    i = lax.bitcast_convert_type(t, jnp.int32)
    ex = lax.shift_right_logical(i, 23) - 127
    m = lax.bitcast_convert_type((i & 0x7FFFFF) | 0x3F800000, jnp.float32)
    big = m > _SQRT2
    m = jnp.where(big, m * 0.5, m)
    exf = ex.astype(jnp.float32) + jnp.where(big, 1.0, 0.0)
    f = m - 1.0
    z = f * f
    p = jnp.full_like(f, _LOG_COEFFS[0])
    for c in _LOG_COEFFS[1:]:
        p = p * f + c
    logt = f * z * p - 0.5 * z + f + exf * _LN2
    return jnp.where(x > 20.0, x, logt)


def kernel(idxs, U, V):
    B = idxs.shape[0]
    D = U.shape[1]
    R = 2 * B
    rows_per_w = R // _NW
    nbatch = rows_per_w // _RING
    flat = idxs.astype(jnp.int32).reshape(-1)
    order = jnp.argsort(flat).astype(jnp.int32)
    sidx = flat[order].reshape(_NW, rows_per_w)
    pos = order.reshape(_NW, rows_per_w)
    UT = U.T
    VT = V.T

    mesh = plsc.VectorSubcoreMesh(core_axis_name="c", subcore_axis_name="s")

    @functools.partial(
        pl.kernel,
        out_type=jax.ShapeDtypeStruct((R, 2 * D), jnp.float32),
        mesh=mesh,
        compiler_params=pltpu.CompilerParams(use_tc_tiling_on_sc=True,
                                             needs_layout_passes=False),
        scratch_types=[
            pltpu.VMEM((rows_per_w + _L,), jnp.int32),
            pltpu.VMEM((rows_per_w + _L,), jnp.int32),
            pltpu.VMEM((D, _BLK), jnp.float32),
            pltpu.VMEM((D, _BLK), jnp.float32),
            pltpu.VMEM((_RING, 2 * D), jnp.float32),
            pltpu.VMEM((8, _RING), jnp.int32),
            pltpu.SemaphoreType.DMA,
        ],
    )
    def run(sidx_hbm, pos_hbm, ut_hbm, vt_hbm, out_hbm,
            idx_v, pos_v, ub, vb, ring, ring_pos, so):
        wid = lax.axis_index("s") * 2 + lax.axis_index("c")
        pltpu.sync_copy(sidx_hbm.at[wid], idx_v.at[pl.ds(0, rows_per_w)])
        pltpu.sync_copy(pos_hbm.at[wid], pos_v.at[pl.ds(0, rows_per_w)])

        lanes = lax.iota(jnp.int32, _L)
        lane0 = lanes == 0
        zeros = jnp.zeros_like(lanes)

        def batch_body(g, b_cur):
            def row_body(t, b_in):
                jj = g * _RING + t
                k = idx_v[pl.ds(jj, _L)][0]
                bn = k // _BLK

                @pl.when(bn != b_in)
                def _():
                    pltpu.sync_copy(ut_hbm.at[:, pl.ds(bn * _BLK, _BLK)], ub)
                    pltpu.sync_copy(vt_hbm.at[:, pl.ds(bn * _BLK, _BLK)], vb)

                col = jnp.full_like(lanes, k - bn * _BLK)
                pvec = pos_v[pl.ds(jj, _L)]
                for l in range(D // _L):
                    rvec = lanes + l * _L
                    uv = plsc.load_gather(ub, [rvec, col])
                    ring[t, pl.ds(l * _L, _L)] = uv
                    vv = plsc.load_gather(vb, [rvec, col])
                    ring[t, pl.ds(D + l * _L, _L)] = _softplus16(vv)
                plsc.store_scatter(
                    ring_pos, [zeros, jnp.full_like(lanes, t)], pvec, mask=lane0)
                return bn

            b_out = lax.fori_loop(0, _RING, row_body, b_cur)
            pltpu.async_copy(ring, out_hbm.at[ring_pos.at[0]], so).wait()
            return b_out

        lax.fori_loop(0, nbatch, batch_body, jnp.int32(-1))

    out = run(sidx, pos, UT, VT)
    return out.reshape(B, 2, 2, D)


# speedup vs baseline: 2.9449x; 1.4242x over previous
import functools

import jax
import jax.numpy as jnp
from jax import lax
from jax.experimental import pallas as pl
from jax.experimental.pallas import tpu as pltpu
from jax.experimental.pallas import tpu_sc as plsc

_L = 16
_NW = 32
_BLK = 256
_RING = 128

_LOG_COEFFS = (
    7.0376836292e-2, -1.1514610310e-1, 1.1676998740e-1,
    -1.2420140846e-1, 1.4249322787e-1, -1.6668057665e-1,
    1.9999714748e-1, -2.4999993993e-1, 3.3333331174e-1,
)
_LN2 = 0.6931471805599453
_SQRT2 = 1.41421356


def _softplus16(x):
    e = jnp.exp(jnp.minimum(x, 20.0))
    t = 1.0 + e
    i = lax.bitcast_convert_type(t, jnp.int32)
    ex = lax.shift_right_logical(i, 23) - 127
    m = lax.bitcast_convert_type((i & 0x7FFFFF) | 0x3F800000, jnp.float32)
    big = m > _SQRT2
    m = jnp.where(big, m * 0.5, m)
    exf = ex.astype(jnp.float32) + jnp.where(big, 1.0, 0.0)
    f = m - 1.0
    z = f * f
    p = jnp.full_like(f, _LOG_COEFFS[0])
    for c in _LOG_COEFFS[1:]:
        p = p * f + c
    logt = f * z * p - 0.5 * z + f + exf * _LN2
    return jnp.where(x > 20.0, x, logt)


def kernel(idxs, U, V):
    B = idxs.shape[0]
    D = U.shape[1]
    NE = U.shape[0]
    R = 2 * B
    rows_per_w = R // _NW
    nbatch = rows_per_w // _RING
    bmax = (NE - 1) // _BLK
    flat = idxs.astype(jnp.int32).reshape(-1)
    order = jnp.argsort(flat).astype(jnp.int32)
    sidx = flat[order].reshape(_NW, rows_per_w)
    pos = order.reshape(_NW, rows_per_w)
    UT = U.T
    VT = V.T

    mesh = plsc.VectorSubcoreMesh(core_axis_name="c", subcore_axis_name="s")

    @functools.partial(
        pl.kernel,
        out_type=jax.ShapeDtypeStruct((R, 2 * D), jnp.float32),
        mesh=mesh,
        compiler_params=pltpu.CompilerParams(use_tc_tiling_on_sc=True,
                                             needs_layout_passes=False),
        scratch_types=[
            pltpu.VMEM((rows_per_w + _L,), jnp.int32),
            pltpu.VMEM((rows_per_w + _L,), jnp.int32),
            pltpu.VMEM((D, _BLK), jnp.float32),
            pltpu.VMEM((D, _BLK), jnp.float32),
            pltpu.VMEM((D, _BLK), jnp.float32),
            pltpu.VMEM((D, _BLK), jnp.float32),
            pltpu.VMEM((_RING, 2 * D), jnp.float32),
            pltpu.VMEM((8, _RING), jnp.int32),
            pltpu.SemaphoreType.DMA,
            pltpu.SemaphoreType.DMA,
        ],
    )
    def run(sidx_hbm, pos_hbm, ut_hbm, vt_hbm, out_hbm,
            idx_v, pos_v, ubA, vbA, ubB, vbB, ring, ring_pos, spf, so):
        wid = lax.axis_index("s") * 2 + lax.axis_index("c")
        pltpu.sync_copy(sidx_hbm.at[wid], idx_v.at[pl.ds(0, rows_per_w)])
        pltpu.sync_copy(pos_hbm.at[wid], pos_v.at[pl.ds(0, rows_per_w)])

        lanes = lax.iota(jnp.int32, _L)
        lane0 = lanes == 0
        zeros = jnp.zeros_like(lanes)

        def load_sync(b, ub, vb):
            pltpu.sync_copy(ut_hbm.at[:, pl.ds(b * _BLK, _BLK)], ub)
            pltpu.sync_copy(vt_hbm.at[:, pl.ds(b * _BLK, _BLK)], vb)

        def load_async(b, ub, vb):
            pltpu.async_copy(ut_hbm.at[:, pl.ds(b * _BLK, _BLK)], ub, spf)
            pltpu.async_copy(vt_hbm.at[:, pl.ds(b * _BLK, _BLK)], vb, spf)

        def drain_prefetch():
            pltpu.make_async_copy(ut_hbm.at[:, pl.ds(0, _BLK)], ubA, spf).wait()
            pltpu.make_async_copy(vt_hbm.at[:, pl.ds(0, _BLK)], vbA, spf).wait()

        b0 = idx_v[pl.ds(0, _L)][0] // _BLK

        @pl.when((b0 & 1) == 0)
        def _():
            load_sync(b0, ubA, vbA)

        @pl.when((b0 & 1) == 1)
        def _():
            load_sync(b0, ubB, vbB)

        @pl.when(((b0 + 1) & 1) == 0)
        def _():
            load_async(jnp.minimum(b0 + 1, bmax), ubA, vbA)

        @pl.when(((b0 + 1) & 1) == 1)
        def _():
            load_async(jnp.minimum(b0 + 1, bmax), ubB, vbB)

        def batch_body(g, b_cur):
            def row_body(t, b_in):
                jj = g * _RING + t
                k = idx_v[pl.ds(jj, _L)][0]
                bn = k // _BLK
                sw = bn != b_in
                miss = jnp.logical_and(sw, bn != b_in + 1)
                p0 = (bn & 1) == 0
                q0 = ((bn + 1) & 1) == 0
                pf = jnp.minimum(bn + 1, bmax)

                @pl.when(sw)
                def _():
                    drain_prefetch()

                @pl.when(jnp.logical_and(miss, p0))
                def _():
                    load_sync(bn, ubA, vbA)

                @pl.when(jnp.logical_and(miss, jnp.logical_not(p0)))
                def _():
                    load_sync(bn, ubB, vbB)

                @pl.when(jnp.logical_and(sw, q0))
                def _():
                    load_async(pf, ubA, vbA)

                @pl.when(jnp.logical_and(sw, jnp.logical_not(q0)))
                def _():
                    load_async(pf, ubB, vbB)

                col = jnp.full_like(lanes, k - bn * _BLK)
                pvec = pos_v[pl.ds(jj, _L)]

                def extract(ub, vb):
                    for l in range(D // _L):
                        rvec = lanes + l * _L
                        uv = plsc.load_gather(ub, [rvec, col])
                        ring[t, pl.ds(l * _L, _L)] = uv
                        vv = plsc.load_gather(vb, [rvec, col])
                        ring[t, pl.ds(D + l * _L, _L)] = _softplus16(vv)

                @pl.when(p0)
                def _():
                    extract(ubA, vbA)

                @pl.when(jnp.logical_not(p0))
                def _():
                    extract(ubB, vbB)

                plsc.store_scatter(
                    ring_pos, [zeros, jnp.full_like(lanes, t)], pvec, mask=lane0)
                return bn

            b_out = lax.fori_loop(0, _RING, row_body, b_cur)
            pltpu.async_copy(ring, out_hbm.at[ring_pos.at[0]], so).wait()
            return b_out

        lax.fori_loop(0, nbatch, batch_body, b0)
        drain_prefetch()

    out = run(sidx, pos, UT, VT)
    return out.reshape(B, 2, 2, D)


# speedup vs baseline: 3.0166x; 1.0243x over previous
import functools

import jax
import jax.numpy as jnp
from jax import lax
from jax.experimental import pallas as pl
from jax.experimental.pallas import tpu as pltpu
from jax.experimental.pallas import tpu_sc as plsc

_L = 16
_NW = 32
_BLK = 384
_RING = 128

_LOG_COEFFS = (
    7.0376836292e-2, -1.1514610310e-1, 1.1676998740e-1,
    -1.2420140846e-1, 1.4249322787e-1, -1.6668057665e-1,
    1.9999714748e-1, -2.4999993993e-1, 3.3333331174e-1,
)
_LN2 = 0.6931471805599453
_SQRT2 = 1.41421356


def _softplus16(x):
    e = jnp.exp(jnp.minimum(x, 20.0))
    t = 1.0 + e
    i = lax.bitcast_convert_type(t, jnp.int32)
    ex = lax.shift_right_logical(i, 23) - 127
    m = lax.bitcast_convert_type((i & 0x7FFFFF) | 0x3F800000, jnp.float32)
    big = m > _SQRT2
    m = jnp.where(big, m * 0.5, m)
    exf = ex.astype(jnp.float32) + jnp.where(big, 1.0, 0.0)
    f = m - 1.0
    z = f * f
    p = jnp.full_like(f, _LOG_COEFFS[0])
    for c in _LOG_COEFFS[1:]:
        p = p * f + c
    logt = f * z * p - 0.5 * z + f + exf * _LN2
    return jnp.where(x > 20.0, x, logt)


def kernel(idxs, U, V):
    B = idxs.shape[0]
    D = U.shape[1]
    NE = U.shape[0]
    R = 2 * B
    rows_per_w = R // _NW
    nbatch = rows_per_w // _RING
    bmax = (NE - 1) // _BLK
    flat = idxs.astype(jnp.int32).reshape(-1)
    order = jnp.argsort(flat).astype(jnp.int32)
    sidx = flat[order].reshape(_NW, rows_per_w)
    pos = order.reshape(_NW, rows_per_w)
    UT = U.T
    VT = V.T

    mesh = plsc.VectorSubcoreMesh(core_axis_name="c", subcore_axis_name="s")

    @functools.partial(
        pl.kernel,
        out_type=jax.ShapeDtypeStruct((R, 2 * D), jnp.float32),
        mesh=mesh,
        compiler_params=pltpu.CompilerParams(use_tc_tiling_on_sc=True,
                                             needs_layout_passes=False),
        scratch_types=[
            pltpu.VMEM((rows_per_w + _L,), jnp.int32),
            pltpu.VMEM((rows_per_w + _L,), jnp.int32),
            pltpu.VMEM((D, _BLK), jnp.float32),
            pltpu.VMEM((D, _BLK), jnp.float32),
            pltpu.VMEM((D, _BLK), jnp.float32),
            pltpu.VMEM((D, _BLK), jnp.float32),
            pltpu.VMEM((_RING, 2 * D), jnp.float32),
            pltpu.VMEM((8, _RING), jnp.int32),
            pltpu.SemaphoreType.DMA,
            pltpu.SemaphoreType.DMA,
        ],
    )
    def run(sidx_hbm, pos_hbm, ut_hbm, vt_hbm, out_hbm,
            idx_v, pos_v, ubA, vbA, ubB, vbB, ring, ring_pos, spf, so):
        wid = lax.axis_index("s") * 2 + lax.axis_index("c")
        pltpu.sync_copy(sidx_hbm.at[wid], idx_v.at[pl.ds(0, rows_per_w)])
        pltpu.sync_copy(pos_hbm.at[wid], pos_v.at[pl.ds(0, rows_per_w)])

        lanes = lax.iota(jnp.int32, _L)
        lane0 = lanes == 0
        zeros = jnp.zeros_like(lanes)

        def load_sync(b, ub, vb):
            pltpu.sync_copy(ut_hbm.at[:, pl.ds(b * _BLK, _BLK)], ub)
            pltpu.sync_copy(vt_hbm.at[:, pl.ds(b * _BLK, _BLK)], vb)

        def load_async(b, ub, vb):
            pltpu.async_copy(ut_hbm.at[:, pl.ds(b * _BLK, _BLK)], ub, spf)
            pltpu.async_copy(vt_hbm.at[:, pl.ds(b * _BLK, _BLK)], vb, spf)

        def drain_prefetch():
            pltpu.make_async_copy(ut_hbm.at[:, pl.ds(0, _BLK)], ubA, spf).wait()
            pltpu.make_async_copy(vt_hbm.at[:, pl.ds(0, _BLK)], vbA, spf).wait()

        b0 = idx_v[pl.ds(0, _L)][0] // _BLK

        @pl.when((b0 & 1) == 0)
        def _():
            load_sync(b0, ubA, vbA)

        @pl.when((b0 & 1) == 1)
        def _():
            load_sync(b0, ubB, vbB)

        @pl.when(((b0 + 1) & 1) == 0)
        def _():
            load_async(jnp.minimum(b0 + 1, bmax), ubA, vbA)

        @pl.when(((b0 + 1) & 1) == 1)
        def _():
            load_async(jnp.minimum(b0 + 1, bmax), ubB, vbB)

        def batch_body(g, b_cur):
            def row_body(t, b_in):
                jj = g * _RING + t
                k = idx_v[pl.ds(jj, _L)][0]
                bn = k // _BLK
                sw = bn != b_in
                miss = jnp.logical_and(sw, bn != b_in + 1)
                p0 = (bn & 1) == 0
                q0 = ((bn + 1) & 1) == 0
                pf = jnp.minimum(bn + 1, bmax)

                @pl.when(sw)
                def _():
                    drain_prefetch()

                @pl.when(jnp.logical_and(miss, p0))
                def _():
                    load_sync(bn, ubA, vbA)

                @pl.when(jnp.logical_and(miss, jnp.logical_not(p0)))
                def _():
                    load_sync(bn, ubB, vbB)

                @pl.when(jnp.logical_and(sw, q0))
                def _():
                    load_async(pf, ubA, vbA)

                @pl.when(jnp.logical_and(sw, jnp.logical_not(q0)))
                def _():
                    load_async(pf, ubB, vbB)

                col = jnp.full_like(lanes, k - bn * _BLK)
                pvec = pos_v[pl.ds(jj, _L)]

                def extract(ub, vb):
                    for l in range(D // _L):
                        rvec = lanes + l * _L
                        uv = plsc.load_gather(ub, [rvec, col])
                        ring[t, pl.ds(l * _L, _L)] = uv
                        vv = plsc.load_gather(vb, [rvec, col])
                        ring[t, pl.ds(D + l * _L, _L)] = _softplus16(vv)

                @pl.when(p0)
                def _():
                    extract(ubA, vbA)

                @pl.when(jnp.logical_not(p0))
                def _():
                    extract(ubB, vbB)

                plsc.store_scatter(
                    ring_pos, [zeros, jnp.full_like(lanes, t)], pvec, mask=lane0)
                return bn

            b_out = lax.fori_loop(0, _RING, row_body, b_cur)
            pltpu.async_copy(ring, out_hbm.at[ring_pos.at[0]], so).wait()
            return b_out

        lax.fori_loop(0, nbatch, batch_body, b0)
        drain_prefetch()

    out = run(sidx, pos, UT, VT)
    return out.reshape(B, 2, 2, D)


# speedup vs baseline: 3.2772x; 1.0864x over previous
import functools

import jax
import jax.numpy as jnp
from jax import lax
from jax.experimental import pallas as pl
from jax.experimental.pallas import tpu as pltpu
from jax.experimental.pallas import tpu_sc as plsc

_L = 16
_NW = 32
_BLK = 384
_RING = 128

_LOG_COEFFS = (
    7.0376836292e-2, -1.1514610310e-1, 1.1676998740e-1,
    -1.2420140846e-1, 1.4249322787e-1, -1.6668057665e-1,
    1.9999714748e-1, -2.4999993993e-1, 3.3333331174e-1,
)
_LN2 = 0.6931471805599453
_SQRT2 = 1.41421356


def _softplus16(x):
    e = jnp.exp(jnp.minimum(x, 20.0))
    t = 1.0 + e
    i = lax.bitcast_convert_type(t, jnp.int32)
    ex = lax.shift_right_logical(i, 23) - 127
    m = lax.bitcast_convert_type((i & 0x7FFFFF) | 0x3F800000, jnp.float32)
    big = m > _SQRT2
    m = jnp.where(big, m * 0.5, m)
    exf = ex.astype(jnp.float32) + jnp.where(big, 1.0, 0.0)
    f = m - 1.0
    z = f * f
    p = jnp.full_like(f, _LOG_COEFFS[0])
    for c in _LOG_COEFFS[1:]:
        p = p * f + c
    logt = f * z * p - 0.5 * z + f + exf * _LN2
    return jnp.where(x > 20.0, x, logt)


def kernel(idxs, U, V):
    B = idxs.shape[0]
    D = U.shape[1]
    NE = U.shape[0]
    R = 2 * B
    rows_per_w = R // _NW
    nbatch = rows_per_w // _RING
    bmax = (NE - 1) // _BLK
    flat = idxs.astype(jnp.int32).reshape(-1)
    order = jnp.argsort(flat).astype(jnp.int32)
    sidx = flat[order].reshape(_NW, rows_per_w)
    pos = order.reshape(_NW, rows_per_w)
    UT = U.T
    VT = V.T

    mesh = plsc.VectorSubcoreMesh(core_axis_name="c", subcore_axis_name="s")

    @functools.partial(
        pl.kernel,
        out_type=jax.ShapeDtypeStruct((R, 2 * D), jnp.float32),
        mesh=mesh,
        compiler_params=pltpu.CompilerParams(use_tc_tiling_on_sc=True,
                                             needs_layout_passes=False),
        scratch_types=[
            pltpu.VMEM((rows_per_w + _L,), jnp.int32),
            pltpu.VMEM((rows_per_w + _L,), jnp.int32),
            pltpu.VMEM((D, _BLK), jnp.float32),
            pltpu.VMEM((D, _BLK), jnp.float32),
            pltpu.VMEM((D, _BLK), jnp.float32),
            pltpu.VMEM((D, _BLK), jnp.float32),
            pltpu.VMEM((_RING, 2 * D), jnp.float32),
            pltpu.VMEM((8, _RING), jnp.int32),
            pltpu.SemaphoreType.DMA,
            pltpu.SemaphoreType.DMA,
        ],
    )
    def run(sidx_hbm, pos_hbm, ut_hbm, vt_hbm, out_hbm,
            idx_v, pos_v, ubA, vbA, ubB, vbB, ring, ring_pos, spf, so):
        wid = lax.axis_index("s") * 2 + lax.axis_index("c")
        pltpu.sync_copy(sidx_hbm.at[wid], idx_v.at[pl.ds(0, rows_per_w)])
        pltpu.sync_copy(pos_hbm.at[wid], pos_v.at[pl.ds(0, rows_per_w)])

        lanes = lax.iota(jnp.int32, _L)
        lane0 = lanes == 0
        zeros = jnp.zeros_like(lanes)

        def load_sync(b, ub, vb):
            pltpu.sync_copy(ut_hbm.at[:, pl.ds(b * _BLK, _BLK)], ub)
            pltpu.sync_copy(vt_hbm.at[:, pl.ds(b * _BLK, _BLK)], vb)

        def load_async(b, ub, vb):
            pltpu.async_copy(ut_hbm.at[:, pl.ds(b * _BLK, _BLK)], ub, spf)
            pltpu.async_copy(vt_hbm.at[:, pl.ds(b * _BLK, _BLK)], vb, spf)

        def drain_prefetch():
            pltpu.make_async_copy(ut_hbm.at[:, pl.ds(0, _BLK)], ubA, spf).wait()
            pltpu.make_async_copy(vt_hbm.at[:, pl.ds(0, _BLK)], vbA, spf).wait()

        b0 = idx_v[pl.ds(0, _L)][0] // _BLK

        @pl.when((b0 & 1) == 0)
        def _():
            load_sync(b0, ubA, vbA)

        @pl.when((b0 & 1) == 1)
        def _():
            load_sync(b0, ubB, vbB)

        @pl.when(((b0 + 1) & 1) == 0)
        def _():
            load_async(jnp.minimum(b0 + 1, bmax), ubA, vbA)

        @pl.when(((b0 + 1) & 1) == 1)
        def _():
            load_async(jnp.minimum(b0 + 1, bmax), ubB, vbB)

        def batch_body(g, b_cur):
            def row_body(t, b_in):
                jj = g * _RING + t
                k = idx_v[pl.ds(jj, _L)][0]
                bn = k // _BLK
                sw = bn != b_in
                miss = jnp.logical_and(sw, bn != b_in + 1)
                p0 = (bn & 1) == 0
                q0 = ((bn + 1) & 1) == 0
                pf = jnp.minimum(bn + 1, bmax)

                @pl.when(sw)
                def _():
                    drain_prefetch()

                @pl.when(jnp.logical_and(miss, p0))
                def _():
                    load_sync(bn, ubA, vbA)

                @pl.when(jnp.logical_and(miss, jnp.logical_not(p0)))
                def _():
                    load_sync(bn, ubB, vbB)

                @pl.when(jnp.logical_and(sw, q0))
                def _():
                    load_async(pf, ubA, vbA)

                @pl.when(jnp.logical_and(sw, jnp.logical_not(q0)))
                def _():
                    load_async(pf, ubB, vbB)

                col = jnp.full_like(lanes, k - bn * _BLK)
                pvec = pos_v[pl.ds(jj, _L)]

                def extract(ub, vb):
                    for l in range(D // _L):
                        rvec = lanes + l * _L
                        uv = plsc.load_gather(ub, [rvec, col])
                        ring[t, pl.ds(l * _L, _L)] = uv
                        vv = plsc.load_gather(vb, [rvec, col])
                        ring[t, pl.ds(D + l * _L, _L)] = _softplus16(vv)

                @pl.when(p0)
                def _():
                    extract(ubA, vbA)

                @pl.when(jnp.logical_not(p0))
                def _():
                    extract(ubB, vbB)

                plsc.store_scatter(
                    ring_pos, [zeros, jnp.full_like(lanes, t)], pvec, mask=lane0)
                return bn

            b_out = lax.fori_loop(0, _RING, row_body, b_cur)
            pltpu.async_copy(ring, out_hbm.at[ring_pos.at[0]], so).wait()
            return b_out

        lax.fori_loop(0, nbatch, batch_body, b0)
        drain_prefetch()

    out = run(sidx, pos, UT, VT)
    mins = out[:, :D].reshape(B, 2, D)
    deltas = out[:, D:].reshape(B, 2, D)
    return jnp.stack([mins, deltas], axis=-2)
